# Initial kernel scaffold; baseline (speedup 1.0000x reference)
import jax
import jax.numpy as jnp
from jax.experimental import pallas as pl


def kernel(x, edge_index, batch, W1, b1, W2, b2, Wc, bc):
    raise NotImplementedError("write your pallas kernel here")


# speedup vs baseline: 4.8319x; 4.8319x over previous
import functools

import jax
import jax.numpy as jnp
from jax import lax
from jax.experimental import pallas as pl
from jax.experimental.pallas import tpu as pltpu
from jax.experimental.pallas import tpu_sc as plsc

_N_NODES = 10000
_N_PAD = 10240
_D = 128
_N_GRAPHS = 64
_NC, _NS = 2, 16
_NW = _NC * _NS
_CHUNK = 128
_E_PAD = 327680
_CPT = _E_PAD // (_NW * _CHUNK)
_TILE_ROWS = _N_PAD // _NS
_DUMMY = _N_NODES
_DEGW = 128

_f32 = jnp.float32


def _mesh():
    return plsc.VectorSubcoreMesh(
        core_axis_name="c", subcore_axis_name="s",
        num_cores=_NC, num_subcores=_NS)


@functools.partial(
    pl.kernel,
    out_type=jax.ShapeDtypeStruct((_NC, _N_PAD, _D), _f32),
    mesh=_mesh(),
    scratch_types=[
        pltpu.VMEM((_CHUNK,), jnp.int32),
        pltpu.VMEM((_CHUNK,), jnp.int32),
        pltpu.VMEM((_CHUNK, _D), _f32),
        pltpu.VMEM_SHARED((_N_PAD, _D), _f32),
        pltpu.SemaphoreType.DMA,
    ],
)
def _edge_kernel(src_hbm, dst_hbm, table_hbm, out_hbm,
                 sidx, didx, rows, acc, sem):
    c = lax.axis_index("c")
    s = lax.axis_index("s")
    wid = c * _NS + s

    def zrow(r, carry):
        def zseg(k, carry2):
            rows[r, pl.ds(k * 16, 16)] = jnp.zeros((16,), _f32)
            return carry2
        lax.fori_loop(0, _D // 16, zseg, 0)
        return carry

    lax.fori_loop(0, _CHUNK, zrow, 0)
    for z in range(_TILE_ROWS // _CHUNK):
        pltpu.sync_copy(
            rows, acc.at[pl.ds(s * _TILE_ROWS + z * _CHUNK, _CHUNK)])
    plsc.subcore_barrier()

    def body(j, carry):
        pltpu.sync_copy(src_hbm.at[wid * _CPT + j], sidx)
        pltpu.sync_copy(dst_hbm.at[wid * _CPT + j], didx)
        pltpu.async_copy(table_hbm.at[sidx], rows, sem).wait()
        pltpu.sync_copy(rows, acc.at[didx], add=True)
        return carry

    lax.fori_loop(0, _CPT, body, 0)
    plsc.subcore_barrier()
    pltpu.sync_copy(acc.at[pl.ds(s * _TILE_ROWS, _TILE_ROWS)],
                    out_hbm.at[c, pl.ds(s * _TILE_ROWS, _TILE_ROWS)])


_BLK = 512
_NBLK = _N_PAD // _BLK


def _dinv_of(d0, d1):
    return lax.rsqrt(d0[:, 0] + d1[:, 0] + 1.0)


def _mm1_body(x_ref, w_ref, d0_ref, d1_ref, o_ref):
    dinv = _dinv_of(d0_ref[...], d1_ref[...])
    h = jnp.dot(x_ref[...], w_ref[...], preferred_element_type=_f32)
    o_ref[...] = h * dinv[:, None]


def _mm1(x_pad, W1, d0, d1):
    return pl.pallas_call(
        _mm1_body,
        grid=(_NBLK,),
        in_specs=[
            pl.BlockSpec((_BLK, _D), lambda i: (i, 0)),
            pl.BlockSpec((_D, _D), lambda i: (0, 0)),
            pl.BlockSpec((_BLK, _DEGW), lambda i: (i, 0)),
            pl.BlockSpec((_BLK, _DEGW), lambda i: (i, 0)),
        ],
        out_specs=pl.BlockSpec((_BLK, _D), lambda i: (i, 0)),
        out_shape=jax.ShapeDtypeStruct((_N_PAD, _D), _f32),
    )(x_pad, W1, d0, d1)


def _mm2_body(p0_ref, p1_ref, hp_ref, d0_ref, d1_ref, b_ref, w_ref, o_ref):
    dinv = _dinv_of(d0_ref[...], d1_ref[...])
    pre = dinv[:, None] * (p0_ref[...] + p1_ref[...] + hp_ref[...]) + b_ref[...]
    t = jnp.maximum(pre, 0.0)
    h = jnp.dot(t, w_ref[...], preferred_element_type=_f32)
    o_ref[...] = h * dinv[:, None]


def _mm2(p0, p1, hp, d0, d1, b, W):
    return pl.pallas_call(
        _mm2_body,
        grid=(_NBLK,),
        in_specs=[
            pl.BlockSpec((_BLK, _D), lambda i: (i, 0)),
            pl.BlockSpec((_BLK, _D), lambda i: (i, 0)),
            pl.BlockSpec((_BLK, _D), lambda i: (i, 0)),
            pl.BlockSpec((_BLK, _DEGW), lambda i: (i, 0)),
            pl.BlockSpec((_BLK, _DEGW), lambda i: (i, 0)),
            pl.BlockSpec((1, _D), lambda i: (0, 0)),
            pl.BlockSpec((_D, _D), lambda i: (0, 0)),
        ],
        out_specs=pl.BlockSpec((_BLK, _D), lambda i: (i, 0)),
        out_shape=jax.ShapeDtypeStruct((_N_PAD, _D), _f32),
    )(p0, p1, hp, d0, d1, b, W)


def _final_body(p0_ref, p1_ref, hp_ref, d0_ref, d1_ref, b_ref, batch_ref,
                wc_ref, bc_ref, o_ref, gsum, cnt):
    i = pl.program_id(0)

    @pl.when(i == 0)
    def _init():
        gsum[...] = jnp.zeros_like(gsum)
        cnt[...] = jnp.zeros_like(cnt)

    dinv = _dinv_of(d0_ref[...], d1_ref[...])
    pre = dinv[:, None] * (p0_ref[...] + p1_ref[...] + hp_ref[...]) + b_ref[...]
    h3 = jnp.maximum(pre, 0.0)
    gids = batch_ref[...]
    onehot = (gids == lax.broadcasted_iota(jnp.int32, (1, _N_GRAPHS), 1)
              ).astype(_f32)
    dn = (((0,), (0,)), ((), ()))
    gsum[...] += lax.dot_general(onehot, h3, dn, preferred_element_type=_f32)
    cnt[...] += lax.dot_general(onehot, jnp.ones((_BLK, _D), _f32), dn,
                                preferred_element_type=_f32)

    @pl.when(i == _NBLK - 1)
    def _done():
        g = gsum[...] / jnp.maximum(cnt[...], 1.0)
        o_ref[...] = (jnp.dot(g, wc_ref[...], preferred_element_type=_f32)
                      + bc_ref[...])


def _final(p0, p1, hp, d0, d1, b, batch2d, wc_pad, bc_pad):
    return pl.pallas_call(
        _final_body,
        grid=(_NBLK,),
        in_specs=[
            pl.BlockSpec((_BLK, _D), lambda i: (i, 0)),
            pl.BlockSpec((_BLK, _D), lambda i: (i, 0)),
            pl.BlockSpec((_BLK, _D), lambda i: (i, 0)),
            pl.BlockSpec((_BLK, _DEGW), lambda i: (i, 0)),
            pl.BlockSpec((_BLK, _DEGW), lambda i: (i, 0)),
            pl.BlockSpec((1, _D), lambda i: (0, 0)),
            pl.BlockSpec((_BLK, 1), lambda i: (i, 0)),
            pl.BlockSpec((_D, _D), lambda i: (0, 0)),
            pl.BlockSpec((1, _D), lambda i: (0, 0)),
        ],
        out_specs=pl.BlockSpec((_N_GRAPHS, _D), lambda i: (0, 0)),
        out_shape=jax.ShapeDtypeStruct((_N_GRAPHS, _D), _f32),
        scratch_shapes=[
            pltpu.VMEM((_N_GRAPHS, _D), _f32),
            pltpu.VMEM((_N_GRAPHS, _D), _f32),
        ],
    )(p0, p1, hp, d0, d1, b, batch2d, wc_pad, bc_pad)


def kernel(x, edge_index, batch, W1, b1, W2, b2, Wc, bc):
    n_edges = edge_index.shape[1]
    src = edge_index[0].astype(jnp.int32)
    dst = edge_index[1].astype(jnp.int32)
    pad = jnp.full((_E_PAD - n_edges,), _DUMMY, dtype=jnp.int32)
    src2d = jnp.concatenate([src, pad]).reshape(_E_PAD // _CHUNK, _CHUNK)
    dst2d = jnp.concatenate([dst, pad]).reshape(_E_PAD // _CHUNK, _CHUNK)

    x_pad = jnp.pad(x, ((0, _N_PAD - _N_NODES), (0, 0)))
    batch2d = jnp.pad(batch.astype(jnp.int32), (0, _N_PAD - _N_NODES),
                      constant_values=_N_GRAPHS).reshape(_N_PAD, 1)
    ncls = Wc.shape[1]
    wc_pad = jnp.pad(Wc, ((0, 0), (0, _D - ncls)))
    bc_pad = jnp.pad(bc, (0, _D - ncls)).reshape(1, _D)
    b1r = b1.reshape(1, _D)
    b2r = b2.reshape(1, _D)

    ones_table = jnp.ones((_N_PAD, _D), _f32)
    deg = _edge_kernel(dst2d, dst2d, ones_table)
    d0, d1 = deg[0], deg[1]

    h1p = _mm1(x_pad, W1, d0, d1)
    p1 = _edge_kernel(src2d, dst2d, h1p)
    h2p = _mm2(p1[0], p1[1], h1p, d0, d1, b1r, W2)
    p2 = _edge_kernel(src2d, dst2d, h2p)
    logits = _final(p2[0], p2[1], h2p, d0, d1, b2r, batch2d, wc_pad, bc_pad)
    return logits[:, :ncls]


# speedup vs baseline: 21.2973x; 4.4076x over previous
import functools

import jax
import jax.numpy as jnp
from jax import lax
from jax.experimental import pallas as pl
from jax.experimental.pallas import tpu as pltpu
from jax.experimental.pallas import tpu_sc as plsc

_N_NODES = 10000
_N_PAD = 10240
_D = 128
_N_GRAPHS = 64
_NC, _NS = 2, 16
_NW = _NC * _NS
_CHUNK = 128
_E_PAD = 327680
_CPT = _E_PAD // (_NW * _CHUNK)
_TILE_ROWS = _N_PAD // _NS
_DUMMY = _N_NODES
_DEGW = 128

_f32 = jnp.float32


def _mesh():
    return plsc.VectorSubcoreMesh(
        core_axis_name="c", subcore_axis_name="s",
        num_cores=_NC, num_subcores=_NS)


@functools.partial(
    pl.kernel,
    out_type=jax.ShapeDtypeStruct((_NC, _N_PAD, _D), _f32),
    mesh=_mesh(),
    scratch_types=[
        pltpu.VMEM((_CPT, _CHUNK), jnp.int32),
        pltpu.VMEM((_CHUNK,), jnp.int32),
        pltpu.VMEM((_CHUNK,), jnp.int32),
        pltpu.VMEM((_CHUNK, _D), _f32),
        pltpu.VMEM((_CHUNK, _D), _f32),
        pltpu.VMEM_SHARED((_N_PAD, _D), _f32),
        pltpu.SemaphoreType.DMA,
        pltpu.SemaphoreType.DMA,
    ],
)
def _edge_kernel(src_hbm, dst_hbm, table_hbm, out_hbm,
                 sidx_all, d_a, d_b, r_a, r_b, acc, sem_a, sem_b):
    c = lax.axis_index("c")
    s = lax.axis_index("s")
    wid = c * _NS + s

    def zrow(r, carry):
        def zseg(k, carry2):
            r_a[r, pl.ds(k * 16, 16)] = jnp.zeros((16,), _f32)
            return carry2
        lax.fori_loop(0, _D // 16, zseg, 0)
        return carry

    lax.fori_loop(0, _CHUNK, zrow, 0)
    for z in range(_TILE_ROWS // _CHUNK):
        pltpu.sync_copy(
            r_a, acc.at[pl.ds(s * _TILE_ROWS + z * _CHUNK, _CHUNK)])
    plsc.subcore_barrier()

    pltpu.sync_copy(src_hbm.at[pl.ds(wid * _CPT, _CPT)], sidx_all)
    pltpu.sync_copy(dst_hbm.at[wid * _CPT], d_a)
    pltpu.async_copy(table_hbm.at[sidx_all.at[0]], r_a, sem_a)

    def body(jj, carry):
        j0 = 2 * jj
        j1 = j0 + 1
        j2 = j0 + 2
        pltpu.sync_copy(dst_hbm.at[wid * _CPT + j1], d_b)
        pltpu.async_copy(table_hbm.at[sidx_all.at[j1]], r_b, sem_b)
        pltpu.make_async_copy(table_hbm.at[sidx_all.at[j0]], r_a,
                              sem_a).wait()
        pltpu.sync_copy(r_a, acc.at[d_a], add=True)

        @pl.when(j2 < _CPT)
        def _():
            pltpu.sync_copy(dst_hbm.at[wid * _CPT + j2], d_a)
            pltpu.async_copy(table_hbm.at[sidx_all.at[j2]], r_a, sem_a)

        pltpu.make_async_copy(table_hbm.at[sidx_all.at[j1]], r_b,
                              sem_b).wait()
        pltpu.sync_copy(r_b, acc.at[d_b], add=True)
        return carry

    lax.fori_loop(0, _CPT // 2, body, 0)
    plsc.subcore_barrier()
    pltpu.sync_copy(acc.at[pl.ds(s * _TILE_ROWS, _TILE_ROWS)],
                    out_hbm.at[c, pl.ds(s * _TILE_ROWS, _TILE_ROWS)])


_BLK = 512
_NBLK = _N_PAD // _BLK


def _dinv_of(d0, d1):
    return lax.rsqrt(d0[:, 0] + d1[:, 0] + 1.0)


def _mm1_body(x_ref, w_ref, d0_ref, d1_ref, o_ref):
    dinv = _dinv_of(d0_ref[...], d1_ref[...])
    h = jnp.dot(x_ref[...], w_ref[...], preferred_element_type=_f32)
    o_ref[...] = h * dinv[:, None]


def _mm1(x_pad, W1, d0, d1):
    return pl.pallas_call(
        _mm1_body,
        grid=(_NBLK,),
        in_specs=[
            pl.BlockSpec((_BLK, _D), lambda i: (i, 0)),
            pl.BlockSpec((_D, _D), lambda i: (0, 0)),
            pl.BlockSpec((_BLK, _DEGW), lambda i: (i, 0)),
            pl.BlockSpec((_BLK, _DEGW), lambda i: (i, 0)),
        ],
        out_specs=pl.BlockSpec((_BLK, _D), lambda i: (i, 0)),
        out_shape=jax.ShapeDtypeStruct((_N_PAD, _D), _f32),
    )(x_pad, W1, d0, d1)


def _mm2_body(p0_ref, p1_ref, hp_ref, d0_ref, d1_ref, b_ref, w_ref, o_ref):
    dinv = _dinv_of(d0_ref[...], d1_ref[...])
    pre = dinv[:, None] * (p0_ref[...] + p1_ref[...] + hp_ref[...]) + b_ref[...]
    t = jnp.maximum(pre, 0.0)
    h = jnp.dot(t, w_ref[...], preferred_element_type=_f32)
    o_ref[...] = h * dinv[:, None]


def _mm2(p0, p1, hp, d0, d1, b, W):
    return pl.pallas_call(
        _mm2_body,
        grid=(_NBLK,),
        in_specs=[
            pl.BlockSpec((_BLK, _D), lambda i: (i, 0)),
            pl.BlockSpec((_BLK, _D), lambda i: (i, 0)),
            pl.BlockSpec((_BLK, _D), lambda i: (i, 0)),
            pl.BlockSpec((_BLK, _DEGW), lambda i: (i, 0)),
            pl.BlockSpec((_BLK, _DEGW), lambda i: (i, 0)),
            pl.BlockSpec((1, _D), lambda i: (0, 0)),
            pl.BlockSpec((_D, _D), lambda i: (0, 0)),
        ],
        out_specs=pl.BlockSpec((_BLK, _D), lambda i: (i, 0)),
        out_shape=jax.ShapeDtypeStruct((_N_PAD, _D), _f32),
    )(p0, p1, hp, d0, d1, b, W)


def _final_body(p0_ref, p1_ref, hp_ref, d0_ref, d1_ref, b_ref, batch_ref,
                wc_ref, bc_ref, o_ref, gsum, cnt):
    i = pl.program_id(0)

    @pl.when(i == 0)
    def _init():
        gsum[...] = jnp.zeros_like(gsum)
        cnt[...] = jnp.zeros_like(cnt)

    dinv = _dinv_of(d0_ref[...], d1_ref[...])
    pre = dinv[:, None] * (p0_ref[...] + p1_ref[...] + hp_ref[...]) + b_ref[...]
    h3 = jnp.maximum(pre, 0.0)
    gids = batch_ref[...]
    onehot = (gids == lax.broadcasted_iota(jnp.int32, (1, _N_GRAPHS), 1)
              ).astype(_f32)
    dn = (((0,), (0,)), ((), ()))
    gsum[...] += lax.dot_general(onehot, h3, dn, preferred_element_type=_f32)
    cnt[...] += lax.dot_general(onehot, jnp.ones((_BLK, _D), _f32), dn,
                                preferred_element_type=_f32)

    @pl.when(i == _NBLK - 1)
    def _done():
        g = gsum[...] / jnp.maximum(cnt[...], 1.0)
        o_ref[...] = (jnp.dot(g, wc_ref[...], preferred_element_type=_f32)
                      + bc_ref[...])


def _final(p0, p1, hp, d0, d1, b, batch2d, wc_pad, bc_pad):
    return pl.pallas_call(
        _final_body,
        grid=(_NBLK,),
        in_specs=[
            pl.BlockSpec((_BLK, _D), lambda i: (i, 0)),
            pl.BlockSpec((_BLK, _D), lambda i: (i, 0)),
            pl.BlockSpec((_BLK, _D), lambda i: (i, 0)),
            pl.BlockSpec((_BLK, _DEGW), lambda i: (i, 0)),
            pl.BlockSpec((_BLK, _DEGW), lambda i: (i, 0)),
            pl.BlockSpec((1, _D), lambda i: (0, 0)),
            pl.BlockSpec((_BLK, 1), lambda i: (i, 0)),
            pl.BlockSpec((_D, _D), lambda i: (0, 0)),
            pl.BlockSpec((1, _D), lambda i: (0, 0)),
        ],
        out_specs=pl.BlockSpec((_N_GRAPHS, _D), lambda i: (0, 0)),
        out_shape=jax.ShapeDtypeStruct((_N_GRAPHS, _D), _f32),
        scratch_shapes=[
            pltpu.VMEM((_N_GRAPHS, _D), _f32),
            pltpu.VMEM((_N_GRAPHS, _D), _f32),
        ],
    )(p0, p1, hp, d0, d1, b, batch2d, wc_pad, bc_pad)


def kernel(x, edge_index, batch, W1, b1, W2, b2, Wc, bc):
    n_edges = edge_index.shape[1]
    src = edge_index[0].astype(jnp.int32)
    dst = edge_index[1].astype(jnp.int32)
    pad = _DUMMY + (jnp.arange(_E_PAD - n_edges, dtype=jnp.int32)
                    % (_N_PAD - _N_NODES))
    src2d = jnp.concatenate([src, pad]).reshape(_E_PAD // _CHUNK, _CHUNK)
    dst2d = jnp.concatenate([dst, pad]).reshape(_E_PAD // _CHUNK, _CHUNK)

    x_pad = jnp.pad(x, ((0, _N_PAD - _N_NODES), (0, 0)))
    batch2d = jnp.pad(batch.astype(jnp.int32), (0, _N_PAD - _N_NODES),
                      constant_values=_N_GRAPHS).reshape(_N_PAD, 1)
    ncls = Wc.shape[1]
    wc_pad = jnp.pad(Wc, ((0, 0), (0, _D - ncls)))
    bc_pad = jnp.pad(bc, (0, _D - ncls)).reshape(1, _D)
    b1r = b1.reshape(1, _D)
    b2r = b2.reshape(1, _D)

    ones_table = jnp.ones((_N_PAD, _D), _f32)
    deg = _edge_kernel(dst2d, dst2d, ones_table)
    d0, d1 = deg[0], deg[1]

    h1p = _mm1(x_pad, W1, d0, d1)
    p1 = _edge_kernel(src2d, dst2d, h1p)
    h2p = _mm2(p1[0], p1[1], h1p, d0, d1, b1r, W2)
    p2 = _edge_kernel(src2d, dst2d, h2p)
    logits = _final(p2[0], p2[1], h2p, d0, d1, b2r, batch2d, wc_pad, bc_pad)
    return logits[:, :ncls]


# speedup vs baseline: 28.9285x; 1.3583x over previous
import functools

import jax
import jax.numpy as jnp
from jax import lax
from jax.experimental import pallas as pl
from jax.experimental.pallas import tpu as pltpu
from jax.experimental.pallas import tpu_sc as plsc

_N_NODES = 10000
_N_PAD = 10240
_D = 128
_N_GRAPHS = 64
_NC, _NS = 2, 16
_NW = _NC * _NS
_CHUNK = 128
_E_PAD = 327680
_CPT = _E_PAD // (_NW * _CHUNK)
_TILE_ROWS = _N_PAD // _NS
_DUMMY = _N_NODES
_DEGW = 128

_f32 = jnp.float32


def _mesh():
    return plsc.VectorSubcoreMesh(
        core_axis_name="c", subcore_axis_name="s",
        num_cores=_NC, num_subcores=_NS)


@functools.partial(
    pl.kernel,
    out_type=jax.ShapeDtypeStruct((_NW, _N_PAD), _f32),
    mesh=_mesh(),
    compiler_params=pltpu.CompilerParams(needs_layout_passes=False),
    scratch_types=[
        pltpu.VMEM((_CPT, _CHUNK), jnp.int32),
        pltpu.VMEM((_N_PAD,), _f32),
    ],
)
def _hist_kernel(dst_hbm, out_hbm, idx_v, hist):
    c = lax.axis_index("c")
    s = lax.axis_index("s")
    wid = c * _NS + s

    def zero(g, carry):
        hist[pl.ds(g * 16, 16)] = jnp.zeros((16,), _f32)
        return carry

    lax.fori_loop(0, _N_PAD // 16, zero, 0)
    pltpu.sync_copy(dst_hbm.at[pl.ds(wid * _CPT, _CPT)], idx_v)
    ones = jnp.ones((16,), _f32)

    def body(r, carry):
        def inner(k, carry2):
            idx = idx_v[r, pl.ds(k * 16, 16)]
            plsc.addupdate_scatter(hist, [idx], ones)
            return carry2
        lax.fori_loop(0, _CHUNK // 16, inner, 0)
        return carry

    lax.fori_loop(0, _CPT, body, 0)
    pltpu.sync_copy(hist, out_hbm.at[wid])


@functools.partial(
    pl.kernel,
    out_type=jax.ShapeDtypeStruct((_NC, _N_PAD, _D), _f32),
    mesh=_mesh(),
    scratch_types=[
        pltpu.VMEM((_CPT, _CHUNK), jnp.int32),
        pltpu.VMEM((_CHUNK,), jnp.int32),
        pltpu.VMEM((_CHUNK,), jnp.int32),
        pltpu.VMEM((_CHUNK, _D), _f32),
        pltpu.VMEM((_CHUNK, _D), _f32),
        pltpu.VMEM_SHARED((_N_PAD, _D), _f32),
        pltpu.SemaphoreType.DMA,
        pltpu.SemaphoreType.DMA,
    ],
)
def _edge_kernel(src_hbm, dst_hbm, table_hbm, out_hbm,
                 sidx_all, d_a, d_b, r_a, r_b, acc, sem_a, sem_b):
    c = lax.axis_index("c")
    s = lax.axis_index("s")
    wid = c * _NS + s

    def zrow(r, carry):
        def zseg(k, carry2):
            r_a[r, pl.ds(k * 16, 16)] = jnp.zeros((16,), _f32)
            return carry2
        lax.fori_loop(0, _D // 16, zseg, 0)
        return carry

    lax.fori_loop(0, _CHUNK, zrow, 0)
    for z in range(_TILE_ROWS // _CHUNK):
        pltpu.sync_copy(
            r_a, acc.at[pl.ds(s * _TILE_ROWS + z * _CHUNK, _CHUNK)])
    plsc.subcore_barrier()

    pltpu.sync_copy(src_hbm.at[pl.ds(wid * _CPT, _CPT)], sidx_all)
    pltpu.sync_copy(dst_hbm.at[wid * _CPT], d_a)
    pltpu.async_copy(table_hbm.at[sidx_all.at[0]], r_a, sem_a)

    def body(jj, carry):
        j0 = 2 * jj
        j1 = j0 + 1
        j2 = j0 + 2
        pltpu.sync_copy(dst_hbm.at[wid * _CPT + j1], d_b)
        pltpu.async_copy(table_hbm.at[sidx_all.at[j1]], r_b, sem_b)
        pltpu.make_async_copy(table_hbm.at[sidx_all.at[j0]], r_a,
                              sem_a).wait()
        pltpu.sync_copy(r_a, acc.at[d_a], add=True)

        @pl.when(j2 < _CPT)
        def _():
            pltpu.sync_copy(dst_hbm.at[wid * _CPT + j2], d_a)
            pltpu.async_copy(table_hbm.at[sidx_all.at[j2]], r_a, sem_a)

        pltpu.make_async_copy(table_hbm.at[sidx_all.at[j1]], r_b,
                              sem_b).wait()
        pltpu.sync_copy(r_b, acc.at[d_b], add=True)
        return carry

    lax.fori_loop(0, _CPT // 2, body, 0)
    plsc.subcore_barrier()
    pltpu.sync_copy(acc.at[pl.ds(s * _TILE_ROWS, _TILE_ROWS)],
                    out_hbm.at[c, pl.ds(s * _TILE_ROWS, _TILE_ROWS)])


_BLK = 512
_NBLK = _N_PAD // _BLK


def _dinv_of(dref):
    return lax.rsqrt(jnp.sum(dref, axis=0) + 1.0)


def _mm1_body(x_ref, w_ref, dh_ref, o_ref):
    dinv = _dinv_of(dh_ref[...])
    h = jnp.dot(x_ref[...], w_ref[...], preferred_element_type=_f32)
    o_ref[...] = h * dinv[:, None]


def _mm1(x_pad, W1, dh):
    return pl.pallas_call(
        _mm1_body,
        grid=(_NBLK,),
        in_specs=[
            pl.BlockSpec((_BLK, _D), lambda i: (i, 0)),
            pl.BlockSpec((_D, _D), lambda i: (0, 0)),
            pl.BlockSpec((_NW, _BLK), lambda i: (0, i)),
        ],
        out_specs=pl.BlockSpec((_BLK, _D), lambda i: (i, 0)),
        out_shape=jax.ShapeDtypeStruct((_N_PAD, _D), _f32),
    )(x_pad, W1, dh)


def _mm2_body(p0_ref, p1_ref, hp_ref, dh_ref, b_ref, w_ref, o_ref):
    dinv = _dinv_of(dh_ref[...])
    pre = dinv[:, None] * (p0_ref[...] + p1_ref[...] + hp_ref[...]) + b_ref[...]
    t = jnp.maximum(pre, 0.0)
    h = jnp.dot(t, w_ref[...], preferred_element_type=_f32)
    o_ref[...] = h * dinv[:, None]


def _mm2(p0, p1, hp, dh, b, W):
    return pl.pallas_call(
        _mm2_body,
        grid=(_NBLK,),
        in_specs=[
            pl.BlockSpec((_BLK, _D), lambda i: (i, 0)),
            pl.BlockSpec((_BLK, _D), lambda i: (i, 0)),
            pl.BlockSpec((_BLK, _D), lambda i: (i, 0)),
            pl.BlockSpec((_NW, _BLK), lambda i: (0, i)),
            pl.BlockSpec((1, _D), lambda i: (0, 0)),
            pl.BlockSpec((_D, _D), lambda i: (0, 0)),
        ],
        out_specs=pl.BlockSpec((_BLK, _D), lambda i: (i, 0)),
        out_shape=jax.ShapeDtypeStruct((_N_PAD, _D), _f32),
    )(p0, p1, hp, dh, b, W)


def _final_body(p0_ref, p1_ref, hp_ref, dh_ref, b_ref, batch_ref,
                wc_ref, bc_ref, o_ref, gsum, cnt):
    i = pl.program_id(0)

    @pl.when(i == 0)
    def _init():
        gsum[...] = jnp.zeros_like(gsum)
        cnt[...] = jnp.zeros_like(cnt)

    dinv = _dinv_of(dh_ref[...])
    pre = dinv[:, None] * (p0_ref[...] + p1_ref[...] + hp_ref[...]) + b_ref[...]
    h3 = jnp.maximum(pre, 0.0)
    gids = batch_ref[...]
    onehot = (gids == lax.broadcasted_iota(jnp.int32, (1, _N_GRAPHS), 1)
              ).astype(_f32)
    dn = (((0,), (0,)), ((), ()))
    gsum[...] += lax.dot_general(onehot, h3, dn, preferred_element_type=_f32)
    cnt[...] += lax.dot_general(onehot, jnp.ones((_BLK, _D), _f32), dn,
                                preferred_element_type=_f32)

    @pl.when(i == _NBLK - 1)
    def _done():
        g = gsum[...] / jnp.maximum(cnt[...], 1.0)
        o_ref[...] = (jnp.dot(g, wc_ref[...], preferred_element_type=_f32)
                      + bc_ref[...])


def _final(p0, p1, hp, dh, b, batch2d, wc_pad, bc_pad):
    return pl.pallas_call(
        _final_body,
        grid=(_NBLK,),
        in_specs=[
            pl.BlockSpec((_BLK, _D), lambda i: (i, 0)),
            pl.BlockSpec((_BLK, _D), lambda i: (i, 0)),
            pl.BlockSpec((_BLK, _D), lambda i: (i, 0)),
            pl.BlockSpec((_NW, _BLK), lambda i: (0, i)),
            pl.BlockSpec((1, _D), lambda i: (0, 0)),
            pl.BlockSpec((_BLK, 1), lambda i: (i, 0)),
            pl.BlockSpec((_D, _D), lambda i: (0, 0)),
            pl.BlockSpec((1, _D), lambda i: (0, 0)),
        ],
        out_specs=pl.BlockSpec((_N_GRAPHS, _D), lambda i: (0, 0)),
        out_shape=jax.ShapeDtypeStruct((_N_GRAPHS, _D), _f32),
        scratch_shapes=[
            pltpu.VMEM((_N_GRAPHS, _D), _f32),
            pltpu.VMEM((_N_GRAPHS, _D), _f32),
        ],
    )(p0, p1, hp, dh, b, batch2d, wc_pad, bc_pad)


def kernel(x, edge_index, batch, W1, b1, W2, b2, Wc, bc):
    n_edges = edge_index.shape[1]
    src = edge_index[0].astype(jnp.int32)
    dst = edge_index[1].astype(jnp.int32)
    pad = _DUMMY + (jnp.arange(_E_PAD - n_edges, dtype=jnp.int32)
                    % (_N_PAD - _N_NODES))
    src2d = jnp.concatenate([src, pad]).reshape(_E_PAD // _CHUNK, _CHUNK)
    dst2d = jnp.concatenate([dst, pad]).reshape(_E_PAD // _CHUNK, _CHUNK)

    x_pad = jnp.pad(x, ((0, _N_PAD - _N_NODES), (0, 0)))
    batch2d = jnp.pad(batch.astype(jnp.int32), (0, _N_PAD - _N_NODES),
                      constant_values=_N_GRAPHS).reshape(_N_PAD, 1)
    ncls = Wc.shape[1]
    wc_pad = jnp.pad(Wc, ((0, 0), (0, _D - ncls)))
    bc_pad = jnp.pad(bc, (0, _D - ncls)).reshape(1, _D)
    b1r = b1.reshape(1, _D)
    b2r = b2.reshape(1, _D)

    dh = _hist_kernel(dst2d)
    h1p = _mm1(x_pad, W1, dh)
    p1 = _edge_kernel(src2d, dst2d, h1p)
    h2p = _mm2(p1[0], p1[1], h1p, dh, b1r, W2)
    p2 = _edge_kernel(src2d, dst2d, h2p)
    logits = _final(p2[0], p2[1], h2p, dh, b2r, batch2d, wc_pad, bc_pad)
    return logits[:, :ncls]


# speedup vs baseline: 31.0456x; 1.0732x over previous
import functools

import jax
import jax.numpy as jnp
from jax import lax
from jax.experimental import pallas as pl
from jax.experimental.pallas import tpu as pltpu
from jax.experimental.pallas import tpu_sc as plsc

_N_NODES = 10000
_N_PAD = 10240
_D = 128
_N_GRAPHS = 64
_NC, _NS = 2, 16
_NW = _NC * _NS
_CHUNK = 128
_E_PAD = 327680
_CPT = _E_PAD // (_NW * _CHUNK)
_TILE_ROWS = _N_PAD // _NS
_DUMMY = _N_NODES
_DEGW = 128

_f32 = jnp.float32


def _mesh():
    return plsc.VectorSubcoreMesh(
        core_axis_name="c", subcore_axis_name="s",
        num_cores=_NC, num_subcores=_NS)


@functools.partial(
    pl.kernel,
    out_type=jax.ShapeDtypeStruct((_NW, _N_PAD), _f32),
    mesh=_mesh(),
    compiler_params=pltpu.CompilerParams(needs_layout_passes=False),
    scratch_types=[
        pltpu.VMEM((_CPT, _CHUNK), jnp.int32),
        pltpu.VMEM((_N_PAD,), _f32),
    ],
)
def _hist_kernel(dst_hbm, out_hbm, idx_v, hist):
    c = lax.axis_index("c")
    s = lax.axis_index("s")
    wid = c * _NS + s

    def zero(g, carry):
        hist[pl.ds(g * 16, 16)] = jnp.zeros((16,), _f32)
        return carry

    lax.fori_loop(0, _N_PAD // 16, zero, 0)
    pltpu.sync_copy(dst_hbm.at[pl.ds(wid * _CPT, _CPT)], idx_v)
    ones = jnp.ones((16,), _f32)

    def body(r, carry):
        def inner(k, carry2):
            idx = idx_v[r, pl.ds(k * 16, 16)]
            plsc.addupdate_scatter(hist, [idx], ones)
            return carry2
        lax.fori_loop(0, _CHUNK // 16, inner, 0)
        return carry

    lax.fori_loop(0, _CPT, body, 0)
    pltpu.sync_copy(hist, out_hbm.at[wid])


@functools.partial(
    pl.kernel,
    out_type=jax.ShapeDtypeStruct((_NC, _N_PAD, _D), _f32),
    mesh=_mesh(),
    scratch_types=[
        pltpu.VMEM((_CPT // 2, _CHUNK), jnp.int32),
        pltpu.VMEM((_CPT // 2, _CHUNK), jnp.int32),
        pltpu.VMEM((_CHUNK, _D), _f32),
        pltpu.VMEM((_CHUNK, _D), _f32),
        pltpu.VMEM_SHARED((_N_PAD, _D), _f32),
        pltpu.SemaphoreType.DMA,
        pltpu.SemaphoreType.DMA,
    ],
)
def _edge_kernel(src_hbm, dst_hbm, table_hbm, out_hbm,
                 sidx_all, didx_all, r_a, r_b, acc, sem_a, sem_b):
    c = lax.axis_index("c")
    s = lax.axis_index("s")
    wid = c * _NS + s

    def zrow(r, carry):
        def zseg(k, carry2):
            r_a[r, pl.ds(k * 16, 16)] = jnp.zeros((16,), _f32)
            return carry2
        lax.fori_loop(0, _D // 16, zseg, 0)
        return carry

    lax.fori_loop(0, _CHUNK, zrow, 0)
    for z in range(_TILE_ROWS // _CHUNK):
        pltpu.sync_copy(
            r_a, acc.at[pl.ds(s * _TILE_ROWS + z * _CHUNK, _CHUNK)])
    plsc.subcore_barrier()

    cph = _CPT // 2
    for ph in range(2):
        base = wid * _CPT + ph * cph
        pltpu.sync_copy(src_hbm.at[pl.ds(base, cph)], sidx_all)
        pltpu.sync_copy(dst_hbm.at[pl.ds(base, cph)], didx_all)
        pltpu.async_copy(table_hbm.at[sidx_all.at[0]], r_a, sem_a)

        def body(jj, carry):
            j0 = 2 * jj
            j1 = j0 + 1
            j2 = j0 + 2
            pltpu.async_copy(table_hbm.at[sidx_all.at[j1]], r_b, sem_b)
            pltpu.make_async_copy(table_hbm.at[sidx_all.at[j0]], r_a,
                                  sem_a).wait()
            pltpu.sync_copy(r_a, acc.at[didx_all.at[j0]], add=True)

            @pl.when(j2 < cph)
            def _():
                pltpu.async_copy(table_hbm.at[sidx_all.at[j2]], r_a, sem_a)

            pltpu.make_async_copy(table_hbm.at[sidx_all.at[j1]], r_b,
                                  sem_b).wait()
            pltpu.sync_copy(r_b, acc.at[didx_all.at[j1]], add=True)
            return carry

        lax.fori_loop(0, cph // 2, body, 0)
    plsc.subcore_barrier()
    pltpu.sync_copy(acc.at[pl.ds(s * _TILE_ROWS, _TILE_ROWS)],
                    out_hbm.at[c, pl.ds(s * _TILE_ROWS, _TILE_ROWS)])


_BLK = 512
_NBLK = _N_PAD // _BLK


def _dinv_of(dref):
    return lax.rsqrt(jnp.sum(dref, axis=0) + 1.0)


def _mm1_body(x_ref, w_ref, dh_ref, o_ref):
    dinv = _dinv_of(dh_ref[...])
    h = jnp.dot(x_ref[...], w_ref[...], preferred_element_type=_f32)
    o_ref[...] = h * dinv[:, None]


def _mm1(x_pad, W1, dh):
    return pl.pallas_call(
        _mm1_body,
        grid=(_NBLK,),
        in_specs=[
            pl.BlockSpec((_BLK, _D), lambda i: (i, 0)),
            pl.BlockSpec((_D, _D), lambda i: (0, 0)),
            pl.BlockSpec((_NW, _BLK), lambda i: (0, i)),
        ],
        out_specs=pl.BlockSpec((_BLK, _D), lambda i: (i, 0)),
        out_shape=jax.ShapeDtypeStruct((_N_PAD, _D), _f32),
    )(x_pad, W1, dh)


def _mm2_body(p0_ref, p1_ref, hp_ref, dh_ref, b_ref, w_ref, o_ref):
    dinv = _dinv_of(dh_ref[...])
    pre = dinv[:, None] * (p0_ref[...] + p1_ref[...] + hp_ref[...]) + b_ref[...]
    t = jnp.maximum(pre, 0.0)
    h = jnp.dot(t, w_ref[...], preferred_element_type=_f32)
    o_ref[...] = h * dinv[:, None]


def _mm2(p0, p1, hp, dh, b, W):
    return pl.pallas_call(
        _mm2_body,
        grid=(_NBLK,),
        in_specs=[
            pl.BlockSpec((_BLK, _D), lambda i: (i, 0)),
            pl.BlockSpec((_BLK, _D), lambda i: (i, 0)),
            pl.BlockSpec((_BLK, _D), lambda i: (i, 0)),
            pl.BlockSpec((_NW, _BLK), lambda i: (0, i)),
            pl.BlockSpec((1, _D), lambda i: (0, 0)),
            pl.BlockSpec((_D, _D), lambda i: (0, 0)),
        ],
        out_specs=pl.BlockSpec((_BLK, _D), lambda i: (i, 0)),
        out_shape=jax.ShapeDtypeStruct((_N_PAD, _D), _f32),
    )(p0, p1, hp, dh, b, W)


def _final_body(p0_ref, p1_ref, hp_ref, dh_ref, b_ref, batch_ref,
                wc_ref, bc_ref, o_ref, gsum, cnt):
    i = pl.program_id(0)

    @pl.when(i == 0)
    def _init():
        gsum[...] = jnp.zeros_like(gsum)
        cnt[...] = jnp.zeros_like(cnt)

    dinv = _dinv_of(dh_ref[...])
    pre = dinv[:, None] * (p0_ref[...] + p1_ref[...] + hp_ref[...]) + b_ref[...]
    h3 = jnp.maximum(pre, 0.0)
    gids = batch_ref[...]
    onehot = (gids == lax.broadcasted_iota(jnp.int32, (1, _N_GRAPHS), 1)
              ).astype(_f32)
    dn = (((0,), (0,)), ((), ()))
    gsum[...] += lax.dot_general(onehot, h3, dn, preferred_element_type=_f32)
    cnt[...] += lax.dot_general(onehot, jnp.ones((_BLK, _D), _f32), dn,
                                preferred_element_type=_f32)

    @pl.when(i == _NBLK - 1)
    def _done():
        g = gsum[...] / jnp.maximum(cnt[...], 1.0)
        o_ref[...] = (jnp.dot(g, wc_ref[...], preferred_element_type=_f32)
                      + bc_ref[...])


def _final(p0, p1, hp, dh, b, batch2d, wc_pad, bc_pad):
    return pl.pallas_call(
        _final_body,
        grid=(_NBLK,),
        in_specs=[
            pl.BlockSpec((_BLK, _D), lambda i: (i, 0)),
            pl.BlockSpec((_BLK, _D), lambda i: (i, 0)),
            pl.BlockSpec((_BLK, _D), lambda i: (i, 0)),
            pl.BlockSpec((_NW, _BLK), lambda i: (0, i)),
            pl.BlockSpec((1, _D), lambda i: (0, 0)),
            pl.BlockSpec((_BLK, 1), lambda i: (i, 0)),
            pl.BlockSpec((_D, _D), lambda i: (0, 0)),
            pl.BlockSpec((1, _D), lambda i: (0, 0)),
        ],
        out_specs=pl.BlockSpec((_N_GRAPHS, _D), lambda i: (0, 0)),
        out_shape=jax.ShapeDtypeStruct((_N_GRAPHS, _D), _f32),
        scratch_shapes=[
            pltpu.VMEM((_N_GRAPHS, _D), _f32),
            pltpu.VMEM((_N_GRAPHS, _D), _f32),
        ],
    )(p0, p1, hp, dh, b, batch2d, wc_pad, bc_pad)


def kernel(x, edge_index, batch, W1, b1, W2, b2, Wc, bc):
    n_edges = edge_index.shape[1]
    src = edge_index[0].astype(jnp.int32)
    dst = edge_index[1].astype(jnp.int32)
    pad = _DUMMY + (jnp.arange(_E_PAD - n_edges, dtype=jnp.int32)
                    % (_N_PAD - _N_NODES))
    src2d = jnp.concatenate([src, pad]).reshape(_E_PAD // _CHUNK, _CHUNK)
    dst2d = jnp.concatenate([dst, pad]).reshape(_E_PAD // _CHUNK, _CHUNK)

    x_pad = jnp.pad(x, ((0, _N_PAD - _N_NODES), (0, 0)))
    batch2d = jnp.pad(batch.astype(jnp.int32), (0, _N_PAD - _N_NODES),
                      constant_values=_N_GRAPHS).reshape(_N_PAD, 1)
    ncls = Wc.shape[1]
    wc_pad = jnp.pad(Wc, ((0, 0), (0, _D - ncls)))
    bc_pad = jnp.pad(bc, (0, _D - ncls)).reshape(1, _D)
    b1r = b1.reshape(1, _D)
    b2r = b2.reshape(1, _D)

    dh = _hist_kernel(dst2d)
    h1p = _mm1(x_pad, W1, dh)
    p1 = _edge_kernel(src2d, dst2d, h1p)
    h2p = _mm2(p1[0], p1[1], h1p, dh, b1r, W2)
    p2 = _edge_kernel(src2d, dst2d, h2p)
    logits = _final(p2[0], p2[1], h2p, dh, b2r, batch2d, wc_pad, bc_pad)
    return logits[:, :ncls]


# speedup vs baseline: 32.4340x; 1.0447x over previous
import functools

import jax
import jax.numpy as jnp
from jax import lax
from jax.experimental import pallas as pl
from jax.experimental.pallas import tpu as pltpu
from jax.experimental.pallas import tpu_sc as plsc

_N_NODES = 10000
_N_PAD = 10240
_D = 128
_N_GRAPHS = 64
_NC, _NS = 2, 16
_NW = _NC * _NS
_CHUNK = 128
_E_PAD = 327680
_CPT = _E_PAD // (_NW * _CHUNK)
_TILE_ROWS = _N_PAD // _NS
_DUMMY = _N_NODES
_DEGW = 128

_f32 = jnp.float32


def _mesh():
    return plsc.VectorSubcoreMesh(
        core_axis_name="c", subcore_axis_name="s",
        num_cores=_NC, num_subcores=_NS)


@functools.partial(
    pl.kernel,
    out_type=jax.ShapeDtypeStruct((_NW, _N_PAD), _f32),
    mesh=_mesh(),
    compiler_params=pltpu.CompilerParams(needs_layout_passes=False),
    scratch_types=[
        pltpu.VMEM((_CPT, _CHUNK), jnp.int32),
        pltpu.VMEM((_N_PAD,), _f32),
    ],
)
def _hist_kernel(dst_hbm, out_hbm, idx_v, hist):
    c = lax.axis_index("c")
    s = lax.axis_index("s")
    wid = c * _NS + s

    def zero(g, carry):
        hist[pl.ds(g * 16, 16)] = jnp.zeros((16,), _f32)
        return carry

    lax.fori_loop(0, _N_PAD // 16, zero, 0)
    pltpu.sync_copy(dst_hbm.at[pl.ds(wid * _CPT, _CPT)], idx_v)
    ones = jnp.ones((16,), _f32)

    def body(r, carry):
        def inner(k, carry2):
            idx = idx_v[r, pl.ds(k * 16, 16)]
            plsc.addupdate_scatter(hist, [idx], ones)
            return carry2
        lax.fori_loop(0, _CHUNK // 16, inner, 0)
        return carry

    lax.fori_loop(0, _CPT, body, 0)
    pltpu.sync_copy(hist, out_hbm.at[wid])


@functools.partial(
    pl.kernel,
    out_type=jax.ShapeDtypeStruct((_NC, _N_PAD, _D), _f32),
    mesh=_mesh(),
    scratch_types=[
        pltpu.VMEM((_CPT // 2, _CHUNK), jnp.int32),
        pltpu.VMEM((_CPT // 2, _CHUNK), jnp.int32),
        pltpu.VMEM((_CHUNK, _D), _f32),
        pltpu.VMEM((_CHUNK, _D), _f32),
        pltpu.VMEM_SHARED((_N_PAD, _D), _f32),
        pltpu.SemaphoreType.DMA,
        pltpu.SemaphoreType.DMA,
    ],
)
def _edge_kernel(src_hbm, dst_hbm, table_hbm, out_hbm,
                 sidx_all, didx_all, r_a, r_b, acc, sem_a, sem_b):
    c = lax.axis_index("c")
    s = lax.axis_index("s")
    wid = c * _NS + s

    def zrow(r, carry):
        def zseg(k, carry2):
            r_a[r, pl.ds(k * 16, 16)] = jnp.zeros((16,), _f32)
            return carry2
        lax.fori_loop(0, _D // 16, zseg, 0)
        return carry

    lax.fori_loop(0, _CHUNK, zrow, 0)
    for z in range(_TILE_ROWS // _CHUNK):
        pltpu.sync_copy(
            r_a, acc.at[pl.ds(s * _TILE_ROWS + z * _CHUNK, _CHUNK)])
    plsc.subcore_barrier()

    cph = _CPT // 2
    for ph in range(2):
        base = wid * _CPT + ph * cph
        pltpu.sync_copy(src_hbm.at[pl.ds(base, cph)], sidx_all)
        pltpu.sync_copy(dst_hbm.at[pl.ds(base, cph)], didx_all)
        pltpu.async_copy(table_hbm.at[sidx_all.at[0]], r_a, sem_a)

        def body(jj, carry):
            j0 = 2 * jj
            j1 = j0 + 1
            j2 = j0 + 2
            pltpu.async_copy(table_hbm.at[sidx_all.at[j1]], r_b, sem_b)
            pltpu.make_async_copy(table_hbm.at[sidx_all.at[j0]], r_a,
                                  sem_a).wait()
            pltpu.sync_copy(r_a, acc.at[didx_all.at[j0]], add=True)

            @pl.when(j2 < cph)
            def _():
                pltpu.async_copy(table_hbm.at[sidx_all.at[j2]], r_a, sem_a)

            pltpu.make_async_copy(table_hbm.at[sidx_all.at[j1]], r_b,
                                  sem_b).wait()
            pltpu.sync_copy(r_b, acc.at[didx_all.at[j1]], add=True)
            return carry

        lax.fori_loop(0, cph // 2, body, 0)
    plsc.subcore_barrier()
    pltpu.sync_copy(acc.at[pl.ds(s * _TILE_ROWS, _TILE_ROWS)],
                    out_hbm.at[c, pl.ds(s * _TILE_ROWS, _TILE_ROWS)])


_BLK = 512
_NBLK = _N_PAD // _BLK


def _dinv_of(dref):
    return lax.rsqrt(jnp.sum(dref, axis=0) + 1.0)


def _mm1_body(x_ref, w_ref, dh_ref, o_ref):
    dinv = _dinv_of(dh_ref[...])
    h = jnp.dot(x_ref[...], w_ref[...], preferred_element_type=_f32)
    o_ref[...] = h * dinv[:, None]


def _mm1(x_pad, W1, dh):
    return pl.pallas_call(
        _mm1_body,
        grid=(_NBLK,),
        in_specs=[
            pl.BlockSpec((_BLK, _D), lambda i: (i, 0)),
            pl.BlockSpec((_D, _D), lambda i: (0, 0)),
            pl.BlockSpec((_NW, _BLK), lambda i: (0, i)),
        ],
        out_specs=pl.BlockSpec((_BLK, _D), lambda i: (i, 0)),
        out_shape=jax.ShapeDtypeStruct((_N_PAD, _D), _f32),
    )(x_pad, W1, dh)


def _mm2_body(p0_ref, p1_ref, hp_ref, dh_ref, b_ref, w_ref, o_ref):
    dinv = _dinv_of(dh_ref[...])
    pre = (dinv[:, None] * (p0_ref[0] + p1_ref[0] + hp_ref[...])
           + b_ref[...])
    t = jnp.maximum(pre, 0.0)
    h = jnp.dot(t, w_ref[...], preferred_element_type=_f32)
    o_ref[...] = h * dinv[:, None]


def _mm2(p, hp, dh, b, W):
    return pl.pallas_call(
        _mm2_body,
        grid=(_NBLK,),
        in_specs=[
            pl.BlockSpec((1, _BLK, _D), lambda i: (0, i, 0)),
            pl.BlockSpec((1, _BLK, _D), lambda i: (1, i, 0)),
            pl.BlockSpec((_BLK, _D), lambda i: (i, 0)),
            pl.BlockSpec((_NW, _BLK), lambda i: (0, i)),
            pl.BlockSpec((1, _D), lambda i: (0, 0)),
            pl.BlockSpec((_D, _D), lambda i: (0, 0)),
        ],
        out_specs=pl.BlockSpec((_BLK, _D), lambda i: (i, 0)),
        out_shape=jax.ShapeDtypeStruct((_N_PAD, _D), _f32),
    )(p, p, hp, dh, b, W)


def _final_body(p0_ref, p1_ref, hp_ref, dh_ref, b_ref, batch_ref,
                wc_ref, bc_ref, o_ref, gsum, cnt):
    i = pl.program_id(0)

    @pl.when(i == 0)
    def _init():
        gsum[...] = jnp.zeros_like(gsum)
        cnt[...] = jnp.zeros_like(cnt)

    dinv = _dinv_of(dh_ref[...])
    pre = (dinv[:, None] * (p0_ref[0] + p1_ref[0] + hp_ref[...])
           + b_ref[...])
    h3 = jnp.maximum(pre, 0.0)
    gids = batch_ref[...]
    onehot = (gids == lax.broadcasted_iota(jnp.int32, (1, _N_GRAPHS), 1)
              ).astype(_f32)
    dn = (((0,), (0,)), ((), ()))
    gsum[...] += lax.dot_general(onehot, h3, dn, preferred_element_type=_f32)
    cnt[...] += lax.dot_general(onehot, jnp.ones((_BLK, _D), _f32), dn,
                                preferred_element_type=_f32)

    @pl.when(i == _NBLK - 1)
    def _done():
        g = gsum[...] / jnp.maximum(cnt[...], 1.0)
        o_ref[...] = (jnp.dot(g, wc_ref[...], preferred_element_type=_f32)
                      + bc_ref[...])


def _final(p, hp, dh, b, batch2d, wc_pad, bc_pad):
    return pl.pallas_call(
        _final_body,
        grid=(_NBLK,),
        in_specs=[
            pl.BlockSpec((1, _BLK, _D), lambda i: (0, i, 0)),
            pl.BlockSpec((1, _BLK, _D), lambda i: (1, i, 0)),
            pl.BlockSpec((_BLK, _D), lambda i: (i, 0)),
            pl.BlockSpec((_NW, _BLK), lambda i: (0, i)),
            pl.BlockSpec((1, _D), lambda i: (0, 0)),
            pl.BlockSpec((_BLK, 1), lambda i: (i, 0)),
            pl.BlockSpec((_D, _D), lambda i: (0, 0)),
            pl.BlockSpec((1, _D), lambda i: (0, 0)),
        ],
        out_specs=pl.BlockSpec((_N_GRAPHS, _D), lambda i: (0, 0)),
        out_shape=jax.ShapeDtypeStruct((_N_GRAPHS, _D), _f32),
        scratch_shapes=[
            pltpu.VMEM((_N_GRAPHS, _D), _f32),
            pltpu.VMEM((_N_GRAPHS, _D), _f32),
        ],
    )(p, p, hp, dh, b, batch2d, wc_pad, bc_pad)


def kernel(x, edge_index, batch, W1, b1, W2, b2, Wc, bc):
    n_edges = edge_index.shape[1]
    src = edge_index[0].astype(jnp.int32)
    dst = edge_index[1].astype(jnp.int32)
    pad = _DUMMY + (jnp.arange(_E_PAD - n_edges, dtype=jnp.int32)
                    % (_N_PAD - _N_NODES))
    src2d = jnp.concatenate([src, pad]).reshape(_E_PAD // _CHUNK, _CHUNK)
    dst2d = jnp.concatenate([dst, pad]).reshape(_E_PAD // _CHUNK, _CHUNK)

    x_pad = jnp.pad(x, ((0, _N_PAD - _N_NODES), (0, 0)))
    batch2d = jnp.pad(batch.astype(jnp.int32), (0, _N_PAD - _N_NODES),
                      constant_values=_N_GRAPHS).reshape(_N_PAD, 1)
    ncls = Wc.shape[1]
    wc_pad = jnp.pad(Wc, ((0, 0), (0, _D - ncls)))
    bc_pad = jnp.pad(bc, (0, _D - ncls)).reshape(1, _D)
    b1r = b1.reshape(1, _D)
    b2r = b2.reshape(1, _D)

    dh = _hist_kernel(dst2d)
    h1p = _mm1(x_pad, W1, dh)
    p1 = _edge_kernel(src2d, dst2d, h1p)
    h2p = _mm2(p1, h1p, dh, b1r, W2)
    p2 = _edge_kernel(src2d, dst2d, h2p)
    logits = _final(p2, h2p, dh, b2r, batch2d, wc_pad, bc_pad)
    return logits[:, :ncls]


# speedup vs baseline: 32.9577x; 1.0161x over previous
import functools

import jax
import jax.numpy as jnp
from jax import lax
from jax.experimental import pallas as pl
from jax.experimental.pallas import tpu as pltpu
from jax.experimental.pallas import tpu_sc as plsc

_N_NODES = 10000
_N_PAD = 10240
_D = 128
_N_GRAPHS = 64
_NC, _NS = 2, 16
_NW = _NC * _NS
_CHUNK = 128
_E_PAD = 327680
_CPT = _E_PAD // (_NW * _CHUNK)
_TILE_ROWS = _N_PAD // _NS
_DUMMY = _N_NODES
_NCLS = 10

_f32 = jnp.float32


def _mesh():
    return plsc.VectorSubcoreMesh(
        core_axis_name="c", subcore_axis_name="s",
        num_cores=_NC, num_subcores=_NS)


@functools.partial(
    pl.kernel,
    out_type=jax.ShapeDtypeStruct((_NW, _N_PAD), _f32),
    mesh=_mesh(),
    compiler_params=pltpu.CompilerParams(needs_layout_passes=False),
    scratch_types=[
        pltpu.VMEM((_CPT, _CHUNK), jnp.int32),
        pltpu.VMEM((_N_PAD,), _f32),
    ],
)
def _hist_kernel(edges_hbm, out_hbm, idx_v, hist):
    c = lax.axis_index("c")
    s = lax.axis_index("s")
    wid = c * _NS + s

    def zero(g, carry):
        hist[pl.ds(g * 16, 16)] = jnp.zeros((16,), _f32)
        return carry

    lax.fori_loop(0, _N_PAD // 16, zero, 0)
    pltpu.sync_copy(edges_hbm.at[1, pl.ds(wid * _CPT, _CPT)], idx_v)
    ones = jnp.ones((16,), _f32)

    def body(r, carry):
        def inner(k, carry2):
            idx = idx_v[r, pl.ds(k * 16, 16)]
            plsc.addupdate_scatter(hist, [idx], ones)
            return carry2
        lax.fori_loop(0, _CHUNK // 16, inner, 0)
        return carry

    lax.fori_loop(0, _CPT, body, 0)
    pltpu.sync_copy(hist, out_hbm.at[wid])


@functools.partial(
    pl.kernel,
    out_type=jax.ShapeDtypeStruct((_NC, _N_PAD, _D), _f32),
    mesh=_mesh(),
    scratch_types=[
        pltpu.VMEM((_CPT // 2, _CHUNK), jnp.int32),
        pltpu.VMEM((_CPT // 2, _CHUNK), jnp.int32),
        pltpu.VMEM((_CHUNK, _D), _f32),
        pltpu.VMEM((_CHUNK, _D), _f32),
        pltpu.VMEM_SHARED((_N_PAD, _D), _f32),
        pltpu.SemaphoreType.DMA,
        pltpu.SemaphoreType.DMA,
    ],
)
def _edge_kernel(edges_hbm, table_hbm, out_hbm,
                 sidx_all, didx_all, r_a, r_b, acc, sem_a, sem_b):
    c = lax.axis_index("c")
    s = lax.axis_index("s")
    wid = c * _NS + s

    def zrow(r, carry):
        def zseg(k, carry2):
            r_a[r, pl.ds(k * 16, 16)] = jnp.zeros((16,), _f32)
            return carry2
        lax.fori_loop(0, _D // 16, zseg, 0)
        return carry

    lax.fori_loop(0, _CHUNK, zrow, 0)
    for z in range(_TILE_ROWS // _CHUNK):
        pltpu.sync_copy(
            r_a, acc.at[pl.ds(s * _TILE_ROWS + z * _CHUNK, _CHUNK)])
    plsc.subcore_barrier()

    cph = _CPT // 2
    for ph in range(2):
        base = wid * _CPT + ph * cph
        pltpu.sync_copy(edges_hbm.at[0, pl.ds(base, cph)], sidx_all)
        pltpu.sync_copy(edges_hbm.at[1, pl.ds(base, cph)], didx_all)
        pltpu.async_copy(table_hbm.at[sidx_all.at[0]], r_a, sem_a)

        def body(jj, carry):
            j0 = 2 * jj
            j1 = j0 + 1
            j2 = j0 + 2
            pltpu.async_copy(table_hbm.at[sidx_all.at[j1]], r_b, sem_b)
            pltpu.make_async_copy(table_hbm.at[sidx_all.at[j0]], r_a,
                                  sem_a).wait()
            pltpu.sync_copy(r_a, acc.at[didx_all.at[j0]], add=True)

            @pl.when(j2 < cph)
            def _():
                pltpu.async_copy(table_hbm.at[sidx_all.at[j2]], r_a, sem_a)

            pltpu.make_async_copy(table_hbm.at[sidx_all.at[j1]], r_b,
                                  sem_b).wait()
            pltpu.sync_copy(r_b, acc.at[didx_all.at[j1]], add=True)
            return carry

        lax.fori_loop(0, cph // 2, body, 0)
    plsc.subcore_barrier()
    pltpu.sync_copy(acc.at[pl.ds(s * _TILE_ROWS, _TILE_ROWS)],
                    out_hbm.at[c, pl.ds(s * _TILE_ROWS, _TILE_ROWS)])


_BLK = 512
_NBLK = _N_PAD // _BLK


def _dinv_of(dref):
    return lax.rsqrt(jnp.sum(dref, axis=0) + 1.0)


def _mm1_body(x_ref, w_ref, dh_ref, o_ref):
    dinv = _dinv_of(dh_ref[...])
    h = jnp.dot(x_ref[...], w_ref[...], preferred_element_type=_f32)
    o_ref[...] = h * dinv[:, None]


def _mm1(x_pad, W1, dh):
    return pl.pallas_call(
        _mm1_body,
        grid=(_NBLK,),
        in_specs=[
            pl.BlockSpec((_BLK, _D), lambda i: (i, 0)),
            pl.BlockSpec((_D, _D), lambda i: (0, 0)),
            pl.BlockSpec((_NW, _BLK), lambda i: (0, i)),
        ],
        out_specs=pl.BlockSpec((_BLK, _D), lambda i: (i, 0)),
        out_shape=jax.ShapeDtypeStruct((_N_PAD, _D), _f32),
    )(x_pad, W1, dh)


def _mm2_body(p0_ref, p1_ref, hp_ref, dh_ref, b_ref, w_ref, o_ref):
    dinv = _dinv_of(dh_ref[...])
    pre = (dinv[:, None] * (p0_ref[0] + p1_ref[0] + hp_ref[...])
           + b_ref[...])
    t = jnp.maximum(pre, 0.0)
    h = jnp.dot(t, w_ref[...], preferred_element_type=_f32)
    o_ref[...] = h * dinv[:, None]


def _mm2(p, hp, dh, b, W):
    return pl.pallas_call(
        _mm2_body,
        grid=(_NBLK,),
        in_specs=[
            pl.BlockSpec((1, _BLK, _D), lambda i: (0, i, 0)),
            pl.BlockSpec((1, _BLK, _D), lambda i: (1, i, 0)),
            pl.BlockSpec((_BLK, _D), lambda i: (i, 0)),
            pl.BlockSpec((_NW, _BLK), lambda i: (0, i)),
            pl.BlockSpec((1, _D), lambda i: (0, 0)),
            pl.BlockSpec((_D, _D), lambda i: (0, 0)),
        ],
        out_specs=pl.BlockSpec((_BLK, _D), lambda i: (i, 0)),
        out_shape=jax.ShapeDtypeStruct((_N_PAD, _D), _f32),
    )(p, p, hp, dh, b, W)


def _final_body(p0_ref, p1_ref, hp_ref, dh_ref, b_ref, batch_ref,
                wc_ref, bc_ref, o_ref, gsum, cnt):
    i = pl.program_id(0)

    @pl.when(i == 0)
    def _init():
        gsum[...] = jnp.zeros_like(gsum)
        cnt[...] = jnp.zeros_like(cnt)

    dinv = _dinv_of(dh_ref[...])
    pre = (dinv[:, None] * (p0_ref[0] + p1_ref[0] + hp_ref[...])
           + b_ref[...])
    h3 = jnp.maximum(pre, 0.0)
    gids = batch_ref[...]
    onehot = (gids == lax.broadcasted_iota(jnp.int32, (1, _N_GRAPHS), 1)
              ).astype(_f32)
    dn = (((0,), (0,)), ((), ()))
    gsum[...] += lax.dot_general(onehot, h3, dn, preferred_element_type=_f32)
    cnt[...] += lax.dot_general(onehot, jnp.ones((_BLK, _D), _f32), dn,
                                preferred_element_type=_f32)

    @pl.when(i == _NBLK - 1)
    def _done():
        g = gsum[...] / jnp.maximum(cnt[...], 1.0)
        full = (jnp.dot(g, wc_ref[...], preferred_element_type=_f32)
                + bc_ref[...])
        o_ref[...] = full[:, :_NCLS]


def _final(p, hp, dh, b, batch2d, wc_pad, bc_pad):
    return pl.pallas_call(
        _final_body,
        grid=(_NBLK,),
        in_specs=[
            pl.BlockSpec((1, _BLK, _D), lambda i: (0, i, 0)),
            pl.BlockSpec((1, _BLK, _D), lambda i: (1, i, 0)),
            pl.BlockSpec((_BLK, _D), lambda i: (i, 0)),
            pl.BlockSpec((_NW, _BLK), lambda i: (0, i)),
            pl.BlockSpec((1, _D), lambda i: (0, 0)),
            pl.BlockSpec((_BLK, 1), lambda i: (i, 0)),
            pl.BlockSpec((_D, _D), lambda i: (0, 0)),
            pl.BlockSpec((1, _D), lambda i: (0, 0)),
        ],
        out_specs=pl.BlockSpec((_N_GRAPHS, _NCLS), lambda i: (0, 0)),
        out_shape=jax.ShapeDtypeStruct((_N_GRAPHS, _NCLS), _f32),
        scratch_shapes=[
            pltpu.VMEM((_N_GRAPHS, _D), _f32),
            pltpu.VMEM((_N_GRAPHS, _D), _f32),
        ],
    )(p, p, hp, dh, b, batch2d, wc_pad, bc_pad)


def kernel(x, edge_index, batch, W1, b1, W2, b2, Wc, bc):
    n_edges = edge_index.shape[1]
    pad = _DUMMY + (jnp.arange(_E_PAD - n_edges, dtype=jnp.int32)
                    % (_N_PAD - _N_NODES))
    edges = jnp.concatenate(
        [edge_index.astype(jnp.int32),
         jnp.broadcast_to(pad, (2, _E_PAD - n_edges))], axis=1,
    ).reshape(2, _E_PAD // _CHUNK, _CHUNK)

    x_pad = jnp.pad(x, ((0, _N_PAD - _N_NODES), (0, 0)))
    batch2d = jnp.pad(batch.astype(jnp.int32), (0, _N_PAD - _N_NODES),
                      constant_values=_N_GRAPHS).reshape(_N_PAD, 1)
    ncls = Wc.shape[1]
    wc_pad = jnp.pad(Wc, ((0, 0), (0, _D - ncls)))
    bc_pad = jnp.pad(bc, (0, _D - ncls)).reshape(1, _D)
    b1r = b1.reshape(1, _D)
    b2r = b2.reshape(1, _D)

    dh = _hist_kernel(edges)
    h1p = _mm1(x_pad, W1, dh)
    p1 = _edge_kernel(edges, h1p)
    h2p = _mm2(p1, h1p, dh, b1r, W2)
    p2 = _edge_kernel(edges, h2p)
    return _final(p2, h2p, dh, b2r, batch2d, wc_pad, bc_pad)


# speedup vs baseline: 34.8663x; 1.0579x over previous
import functools

import jax
import jax.numpy as jnp
from jax import lax
from jax.experimental import pallas as pl
from jax.experimental.pallas import tpu as pltpu
from jax.experimental.pallas import tpu_sc as plsc

_N_NODES = 10000
_N_PAD = 10240
_D = 128
_N_GRAPHS = 64
_NC, _NS = 2, 16
_NW = _NC * _NS
_CHUNK = 128
_E_PAD = 327680
_CPT = _E_PAD // (_NW * _CHUNK)
_TILE_ROWS = _N_PAD // _NS
_DUMMY = _N_NODES
_NCLS = 10

_f32 = jnp.float32


def _mesh():
    return plsc.VectorSubcoreMesh(
        core_axis_name="c", subcore_axis_name="s",
        num_cores=_NC, num_subcores=_NS)


@functools.partial(
    pl.kernel,
    out_type=jax.ShapeDtypeStruct((_NW, _N_PAD), _f32),
    mesh=_mesh(),
    compiler_params=pltpu.CompilerParams(needs_layout_passes=False),
    scratch_types=[
        pltpu.VMEM((_CPT, _CHUNK), jnp.int32),
        pltpu.VMEM((_N_PAD,), _f32),
    ],
)
def _hist_kernel(edges_hbm, out_hbm, idx_v, hist):
    c = lax.axis_index("c")
    s = lax.axis_index("s")
    wid = c * _NS + s

    def zero(g, carry):
        hist[pl.ds(g * 16, 16)] = jnp.zeros((16,), _f32)
        return carry

    lax.fori_loop(0, _N_PAD // 16, zero, 0)
    pltpu.sync_copy(edges_hbm.at[1, pl.ds(wid * _CPT, _CPT)], idx_v)
    ones = jnp.ones((16,), _f32)

    def body(r, carry):
        def inner(k, carry2):
            idx = idx_v[r, pl.ds(k * 16, 16)]
            plsc.addupdate_scatter(hist, [idx], ones)
            return carry2
        lax.fori_loop(0, _CHUNK // 16, inner, 0)
        return carry

    lax.fori_loop(0, _CPT, body, 0)
    pltpu.sync_copy(hist, out_hbm.at[wid])


@functools.partial(
    pl.kernel,
    out_type=jax.ShapeDtypeStruct((_NC, _N_PAD, _D), _f32),
    mesh=_mesh(),
    scratch_types=[
        pltpu.VMEM((_CPT // 2, _CHUNK), jnp.int32),
        pltpu.VMEM((_CPT // 2, _CHUNK), jnp.int32),
        pltpu.VMEM((_CHUNK, _D), _f32),
        pltpu.VMEM((_CHUNK, _D), _f32),
        pltpu.VMEM_SHARED((_N_PAD, _D), _f32),
        pltpu.SemaphoreType.DMA,
        pltpu.SemaphoreType.DMA,
    ],
)
def _edge_kernel(edges_hbm, table_hbm, out_hbm,
                 sidx_all, didx_all, r_a, r_b, acc, sem_a, sem_b):
    c = lax.axis_index("c")
    s = lax.axis_index("s")
    wid = c * _NS + s

    def zrow(r, carry):
        def zseg(k, carry2):
            r_a[r, pl.ds(k * 16, 16)] = jnp.zeros((16,), _f32)
            return carry2
        lax.fori_loop(0, _D // 16, zseg, 0)
        return carry

    lax.fori_loop(0, _CHUNK, zrow, 0)
    for z in range(_TILE_ROWS // _CHUNK):
        pltpu.sync_copy(
            r_a, acc.at[pl.ds(s * _TILE_ROWS + z * _CHUNK, _CHUNK)])
    plsc.subcore_barrier()

    cph = _CPT // 2
    for ph in range(2):
        base = wid * _CPT + ph * cph
        pltpu.sync_copy(edges_hbm.at[0, pl.ds(base, cph)], sidx_all)
        pltpu.sync_copy(edges_hbm.at[1, pl.ds(base, cph)], didx_all)
        pltpu.async_copy(table_hbm.at[sidx_all.at[0]], r_a, sem_a)

        def body(jj, carry):
            j0 = 2 * jj
            j1 = j0 + 1
            j2 = j0 + 2
            pltpu.async_copy(table_hbm.at[sidx_all.at[j1]], r_b, sem_b)
            pltpu.make_async_copy(table_hbm.at[sidx_all.at[j0]], r_a,
                                  sem_a).wait()
            pltpu.sync_copy(r_a, acc.at[didx_all.at[j0]], add=True)

            @pl.when(j2 < cph)
            def _():
                pltpu.async_copy(table_hbm.at[sidx_all.at[j2]], r_a, sem_a)

            pltpu.make_async_copy(table_hbm.at[sidx_all.at[j1]], r_b,
                                  sem_b).wait()
            pltpu.sync_copy(r_b, acc.at[didx_all.at[j1]], add=True)
            return carry

        lax.fori_loop(0, cph // 2, body, 0)
    plsc.subcore_barrier()
    pltpu.sync_copy(acc.at[pl.ds(s * _TILE_ROWS, _TILE_ROWS)],
                    out_hbm.at[c, pl.ds(s * _TILE_ROWS, _TILE_ROWS)])


_BLK = 1024
_NBLK = _N_PAD // _BLK


def _dinv_of(dref):
    return lax.rsqrt(jnp.sum(dref, axis=0) + 1.0)


def _mm1_body(x_ref, w_ref, dh_ref, o_ref):
    dinv = _dinv_of(dh_ref[...])
    h = jnp.dot(x_ref[...], w_ref[...], preferred_element_type=_f32)
    o_ref[...] = h * dinv[:, None]


def _mm1(x_pad, W1, dh):
    return pl.pallas_call(
        _mm1_body,
        grid=(_NBLK,),
        in_specs=[
            pl.BlockSpec((_BLK, _D), lambda i: (i, 0)),
            pl.BlockSpec((_D, _D), lambda i: (0, 0)),
            pl.BlockSpec((_NW, _BLK), lambda i: (0, i)),
        ],
        out_specs=pl.BlockSpec((_BLK, _D), lambda i: (i, 0)),
        out_shape=jax.ShapeDtypeStruct((_N_PAD, _D), _f32),
    )(x_pad, W1, dh)


def _mm2_body(p0_ref, p1_ref, hp_ref, dh_ref, b_ref, w_ref, o_ref):
    dinv = _dinv_of(dh_ref[...])
    pre = (dinv[:, None] * (p0_ref[0] + p1_ref[0] + hp_ref[...])
           + b_ref[...])
    t = jnp.maximum(pre, 0.0)
    h = jnp.dot(t, w_ref[...], preferred_element_type=_f32)
    o_ref[...] = h * dinv[:, None]


def _mm2(p, hp, dh, b, W):
    return pl.pallas_call(
        _mm2_body,
        grid=(_NBLK,),
        in_specs=[
            pl.BlockSpec((1, _BLK, _D), lambda i: (0, i, 0)),
            pl.BlockSpec((1, _BLK, _D), lambda i: (1, i, 0)),
            pl.BlockSpec((_BLK, _D), lambda i: (i, 0)),
            pl.BlockSpec((_NW, _BLK), lambda i: (0, i)),
            pl.BlockSpec((1, _D), lambda i: (0, 0)),
            pl.BlockSpec((_D, _D), lambda i: (0, 0)),
        ],
        out_specs=pl.BlockSpec((_BLK, _D), lambda i: (i, 0)),
        out_shape=jax.ShapeDtypeStruct((_N_PAD, _D), _f32),
    )(p, p, hp, dh, b, W)


def _final_body(p0_ref, p1_ref, hp_ref, dh_ref, b_ref, batch_ref,
                wc_ref, bc_ref, o_ref, gsum, cnt):
    i = pl.program_id(0)

    @pl.when(i == 0)
    def _init():
        gsum[...] = jnp.zeros_like(gsum)
        cnt[...] = jnp.zeros_like(cnt)

    dinv = _dinv_of(dh_ref[...])
    pre = (dinv[:, None] * (p0_ref[0] + p1_ref[0] + hp_ref[...])
           + b_ref[...])
    h3 = jnp.maximum(pre, 0.0)
    gids = batch_ref[...]
    onehot = (gids == lax.broadcasted_iota(jnp.int32, (1, _N_GRAPHS), 1)
              ).astype(_f32)
    dn = (((0,), (0,)), ((), ()))
    gsum[...] += lax.dot_general(onehot, h3, dn, preferred_element_type=_f32)
    cnt[...] += lax.dot_general(onehot, jnp.ones((_BLK, _D), _f32), dn,
                                preferred_element_type=_f32)

    @pl.when(i == _NBLK - 1)
    def _done():
        g = gsum[...] / jnp.maximum(cnt[...], 1.0)
        full = (jnp.dot(g, wc_ref[...], preferred_element_type=_f32)
                + bc_ref[...])
        o_ref[...] = full[:, :_NCLS]


def _final(p, hp, dh, b, batch2d, wc_pad, bc_pad):
    return pl.pallas_call(
        _final_body,
        grid=(_NBLK,),
        in_specs=[
            pl.BlockSpec((1, _BLK, _D), lambda i: (0, i, 0)),
            pl.BlockSpec((1, _BLK, _D), lambda i: (1, i, 0)),
            pl.BlockSpec((_BLK, _D), lambda i: (i, 0)),
            pl.BlockSpec((_NW, _BLK), lambda i: (0, i)),
            pl.BlockSpec((1, _D), lambda i: (0, 0)),
            pl.BlockSpec((_BLK, 1), lambda i: (i, 0)),
            pl.BlockSpec((_D, _D), lambda i: (0, 0)),
            pl.BlockSpec((1, _D), lambda i: (0, 0)),
        ],
        out_specs=pl.BlockSpec((_N_GRAPHS, _NCLS), lambda i: (0, 0)),
        out_shape=jax.ShapeDtypeStruct((_N_GRAPHS, _NCLS), _f32),
        scratch_shapes=[
            pltpu.VMEM((_N_GRAPHS, _D), _f32),
            pltpu.VMEM((_N_GRAPHS, _D), _f32),
        ],
    )(p, p, hp, dh, b, batch2d, wc_pad, bc_pad)


def kernel(x, edge_index, batch, W1, b1, W2, b2, Wc, bc):
    n_edges = edge_index.shape[1]
    pad = _DUMMY + (jnp.arange(_E_PAD - n_edges, dtype=jnp.int32)
                    % (_N_PAD - _N_NODES))
    edges = jnp.concatenate(
        [edge_index.astype(jnp.int32),
         jnp.broadcast_to(pad, (2, _E_PAD - n_edges))], axis=1,
    ).reshape(2, _E_PAD // _CHUNK, _CHUNK)

    x_pad = jnp.pad(x, ((0, _N_PAD - _N_NODES), (0, 0)))
    batch2d = jnp.pad(batch.astype(jnp.int32), (0, _N_PAD - _N_NODES),
                      constant_values=_N_GRAPHS).reshape(_N_PAD, 1)
    ncls = Wc.shape[1]
    wc_pad = jnp.pad(Wc, ((0, 0), (0, _D - ncls)))
    bc_pad = jnp.pad(bc, (0, _D - ncls)).reshape(1, _D)
    b1r = b1.reshape(1, _D)
    b2r = b2.reshape(1, _D)

    dh = _hist_kernel(edges)
    h1p = _mm1(x_pad, W1, dh)
    p1 = _edge_kernel(edges, h1p)
    h2p = _mm2(p1, h1p, dh, b1r, W2)
    p2 = _edge_kernel(edges, h2p)
    return _final(p2, h2p, dh, b2r, batch2d, wc_pad, bc_pad)
